# baseline (speedup 1.0000x reference)
import functools

import jax
import jax.numpy as jnp
from jax import lax
from jax.experimental import pallas as pl
from jax.experimental.pallas import tpu as pltpu
from jax.experimental.pallas import tpu_sc as plsc

DIM = 768
E = 16
K = 8
HD = 96
B = 4
N = 2048
T = B * N
BT = 512
BQ = 512
NBT = T // BT
NQ = N // BQ
EH = E * HD
VW = 128
EHP = E * VW


def _route_proj_kernel(x_ref, wg_ref, wq_ref, kvw_ref, kvb_ref,
                       y_ref, k_ref, v_ref, ws_ref, sq_ref, sz_ref, g_ref,
                       fr_ref, ps_ref, zs_ref):
    x = x_ref[...]
    kv = jnp.dot(x, kvw_ref[...], preferred_element_type=jnp.float32)
    kv = kv + kvb_ref[...]
    lane = jax.lax.broadcasted_iota(jnp.int32, (BT, VW - HD), 1)
    ones_pad = jnp.where(lane == 0, 1.0, 0.0)
    zeros_pad = jnp.zeros((BT, VW - HD), jnp.float32)
    k_ref[...] = jnp.concatenate([kv[:, :HD] * (HD ** -0.5), zeros_pad],
                                 axis=-1)
    v_ref[...] = jnp.concatenate([kv[:, HD:], ones_pad], axis=-1)
    logits = jnp.dot(x, wg_ref[...], preferred_element_type=jnp.float32)
    m = jnp.max(logits, axis=-1, keepdims=True)
    ex = jnp.exp(logits - m)
    se = jnp.sum(ex, axis=-1, keepdims=True)
    p = ex / se
    lse = m + jnp.log(se)
    zs_ref[...] = jnp.broadcast_to(jnp.sum(lse * lse), (1, 1, 8))
    eidx = jax.lax.broadcasted_iota(jnp.int32, (BT, E), 1)
    rank = jnp.zeros((BT, E), jnp.int32)
    for j in range(E):
        pj = p[:, j:j + 1]
        rank = rank + jnp.where((pj > p) | ((pj == p) & (j < eidx)), 1, 0)
    sel = rank < K
    self32 = sel.astype(jnp.float32)
    gm = self32 * p
    g = gm / (jnp.sum(gm, axis=-1, keepdims=True) + 1e-6)
    slot = jnp.zeros((BT, E), jnp.int32)
    for j in range(E):
        sj = jnp.where(sel[:, j:j + 1], 1, 0)
        slot = slot + jnp.where(eidx > j, sj, 0)
    y_ref[...] = jnp.dot(x, wq_ref[...], preferred_element_type=jnp.float32)
    eidxf = eidx.astype(jnp.float32)
    i = pl.program_id(0)
    tgi = jax.lax.broadcasted_iota(jnp.int32, (BT, 1), 0) + i * BT
    tglob = tgi.astype(jnp.float32)
    cols = []
    for kk in range(K):
        sk = self32 * (slot == kk).astype(jnp.float32)
        ws_ref[kk] = sk * g
        etk = jnp.sum(sk * eidxf, axis=-1, keepdims=True)
        cols.append(tglob * E + etk)
    sq = jnp.concatenate(cols, axis=-1).astype(jnp.int32)
    sq_ref[...] = jnp.transpose(sq, (1, 0))
    sz_ref[...] = jnp.where(sel, slot * T + tgi, 0)
    g_ref[...] = g
    fr_ref[0] = jnp.sum(self32, axis=0, keepdims=True)
    ps_ref[0] = jnp.sum(p, axis=0, keepdims=True)


def _attn_moe_kernel(q_ref, ws_ref, k_ref, v_ref, o_ref):
    g = ws_ref[0]
    s = jax.lax.dot_general(q_ref[0], k_ref[0], (((1,), (1,)), ((), ())),
                            preferred_element_type=jnp.float32)
    e = jnp.exp(s)
    oa = jnp.dot(e, v_ref[0], preferred_element_type=jnp.float32)
    gval = jnp.sum(g, axis=-1, keepdims=True)
    ow = oa[:, :HD] * (gval / oa[:, HD:HD + 1])
    o_ref[0] = jnp.concatenate(
        [ow, jnp.zeros((BQ, VW - HD), jnp.float32)], axis=-1)


def _out_proj_kernel(z_ref, g_ref, sel_ref, wo_ref, out_ref):
    wide = jnp.dot(g_ref[...], sel_ref[...],
                   preferred_element_type=jnp.float32)
    out_ref[...] = jnp.dot(wide * z_ref[...], wo_ref[...],
                           preferred_element_type=jnp.float32)


_NW = 32
_CH = 128


def _make_sc_gather(rows):
    rpw = rows // _NW
    nch = rpw // _CH

    @functools.partial(
        pl.kernel,
        mesh=plsc.VectorSubcoreMesh(core_axis_name="c", subcore_axis_name="s"),
        out_type=jax.ShapeDtypeStruct((rows, VW), jnp.float32),
        scratch_types=[
            pltpu.VMEM((_CH,), jnp.int32),
            pltpu.VMEM((_CH, VW), jnp.float32),
            pltpu.SemaphoreType.DMA,
        ],
    )
    def _sc_gather(table_hbm, idx_hbm, out_hbm, idx_v, rows_v, sem):
        wid = lax.axis_index("s") * 2 + lax.axis_index("c")
        base = wid * rpw

        def body(j, carry):
            off = base + j * _CH
            pltpu.sync_copy(idx_hbm.at[pl.ds(off, _CH)], idx_v)
            pltpu.async_copy(table_hbm.at[idx_v], rows_v, sem).wait()
            pltpu.sync_copy(rows_v, out_hbm.at[pl.ds(off, _CH)])
            return carry

        lax.fori_loop(0, nch, body, 0)

    return _sc_gather


_QROWS = K * T
_ZROWS = T * E
_sc_gather_q = _make_sc_gather(_QROWS)
_sc_gather_z = _make_sc_gather(_ZROWS)


def kernel(x, w_gate, Wq, kv_w, kv_b, W_out, task_bh):
    xf = x.reshape(T, DIM)
    wg = w_gate[task_bh]
    wq_flat = jnp.pad(jnp.transpose(Wq, (1, 0, 2)),
                      ((0, 0), (0, 0), (0, VW - HD))).reshape(DIM, EHP)
    wo_flat = W_out.reshape(EH, DIM)
    kvb2 = kv_b.reshape(1, 2 * HD)
    eye_e = jnp.eye(E, dtype=jnp.float32)
    selmat = jnp.repeat(eye_e, VW, axis=1).reshape(E, EHP)
    wo_pad = jnp.pad(W_out, ((0, 0), (0, VW - HD), (0, 0))).reshape(EHP, DIM)

    y, k_, v_, ws, srcq, srcz, g2, fr, ps, zs = pl.pallas_call(
        _route_proj_kernel,
        grid=(NBT,),
        in_specs=[
            pl.BlockSpec((BT, DIM), lambda i: (i, 0)),
            pl.BlockSpec((DIM, E), lambda i: (0, 0)),
            pl.BlockSpec((DIM, EHP), lambda i: (0, 0)),
            pl.BlockSpec((DIM, 2 * HD), lambda i: (0, 0)),
            pl.BlockSpec((1, 2 * HD), lambda i: (0, 0)),
        ],
        out_specs=[
            pl.BlockSpec((BT, EHP), lambda i: (i, 0)),
            pl.BlockSpec((BT, VW), lambda i: (i, 0)),
            pl.BlockSpec((BT, VW), lambda i: (i, 0)),
            pl.BlockSpec((K, BT, E), lambda i: (0, i, 0)),
            pl.BlockSpec((K, BT), lambda i: (0, i)),
            pl.BlockSpec((BT, E), lambda i: (i, 0)),
            pl.BlockSpec((BT, E), lambda i: (i, 0)),
            pl.BlockSpec((1, 1, E), lambda i: (i, 0, 0)),
            pl.BlockSpec((1, 1, E), lambda i: (i, 0, 0)),
            pl.BlockSpec((1, 1, 8), lambda i: (i, 0, 0)),
        ],
        out_shape=[
            jax.ShapeDtypeStruct((T, EHP), jnp.float32),
            jax.ShapeDtypeStruct((T, VW), jnp.float32),
            jax.ShapeDtypeStruct((T, VW), jnp.float32),
            jax.ShapeDtypeStruct((K, T, E), jnp.float32),
            jax.ShapeDtypeStruct((K, T), jnp.int32),
            jax.ShapeDtypeStruct((T, E), jnp.int32),
            jax.ShapeDtypeStruct((T, E), jnp.float32),
            jax.ShapeDtypeStruct((NBT, 1, E), jnp.float32),
            jax.ShapeDtypeStruct((NBT, 1, E), jnp.float32),
            jax.ShapeDtypeStruct((NBT, 1, 8), jnp.float32),
        ],
    )(xf, wg, wq_flat, kv_w, kvb2)

    q = _sc_gather_q(y.reshape(T * E, VW), srcq.reshape(_QROWS))
    q = q.reshape(K, T, VW)

    o = pl.pallas_call(
        _attn_moe_kernel,
        grid=(B, NQ, K),
        in_specs=[
            pl.BlockSpec((1, BQ, VW), lambda b, i, h: (h, b * NQ + i, 0)),
            pl.BlockSpec((1, BQ, E), lambda b, i, h: (h, b * NQ + i, 0)),
            pl.BlockSpec((1, N, VW), lambda b, i, h: (0, b, 0)),
            pl.BlockSpec((1, N, VW), lambda b, i, h: (0, b, 0)),
        ],
        out_specs=pl.BlockSpec((1, BQ, VW), lambda b, i, h: (h, b * NQ + i, 0)),
        out_shape=jax.ShapeDtypeStruct((K, T, VW), jnp.float32),
    )(q, ws, k_.reshape(1, T, VW), v_.reshape(1, T, VW))

    z = _sc_gather_z(o.reshape(K * T, VW), srcz.reshape(_ZROWS))

    out2d = pl.pallas_call(
        _out_proj_kernel,
        grid=(NBT,),
        in_specs=[
            pl.BlockSpec((BT, EHP), lambda i: (i, 0)),
            pl.BlockSpec((BT, E), lambda i: (i, 0)),
            pl.BlockSpec((E, EHP), lambda i: (0, 0)),
            pl.BlockSpec((EHP, DIM), lambda i: (0, 0)),
        ],
        out_specs=pl.BlockSpec((BT, DIM), lambda i: (i, 0)),
        out_shape=jax.ShapeDtypeStruct((T, DIM), jnp.float32),
    )(z.reshape(T, EHP), g2, selmat, wo_pad)

    out = out2d.reshape(B, N, DIM)

    zsum = jnp.sum(zs[:, 0, 0])
    zloss = 0.001 * zsum / T
    freqs = jnp.sum(fr[:, 0, :], axis=0)
    psum = jnp.sum(ps[:, 0, :], axis=0)
    freqs_n = freqs / (jnp.sum(freqs) + 1e-9)
    pm = psum / T
    pm_n = pm / (jnp.sum(pm) + 1e-9)
    switchloss = 0.1 * E * jnp.sum(pm_n * freqs_n)
    aux_loss = zloss + switchloss
    return out, aux_loss


# speedup vs baseline: 4.7061x; 4.7061x over previous
import functools

import jax
import jax.numpy as jnp
from jax import lax
from jax.experimental import pallas as pl
from jax.experimental.pallas import tpu as pltpu
from jax.experimental.pallas import tpu_sc as plsc

DIM = 768
E = 16
K = 8
HD = 96
B = 4
N = 2048
T = B * N
BT = 512
BQ = 512
NBT = T // BT
NQ = N // BQ
EH = E * HD
VW = 128
EHP = E * VW


def _route_proj_kernel(x_ref, wg_ref, wq_ref, kvw_ref, kvb_ref,
                       y_ref, k_ref, v_ref, sq_ref, sz_ref, g_ref,
                       fr_ref, ps_ref, zs_ref):
    x = x_ref[...]
    kv = jnp.dot(x, kvw_ref[...], preferred_element_type=jnp.float32)
    kv = kv + kvb_ref[...]
    lane = jax.lax.broadcasted_iota(jnp.int32, (BT, VW - HD), 1)
    ones_pad = jnp.where(lane == 0, 1.0, 0.0)
    zeros_pad = jnp.zeros((BT, VW - HD), jnp.float32)
    k_ref[...] = jnp.concatenate([kv[:, :HD] * (HD ** -0.5), zeros_pad],
                                 axis=-1)
    v_ref[...] = jnp.concatenate([kv[:, HD:], ones_pad], axis=-1)
    logits = jnp.dot(x, wg_ref[...], preferred_element_type=jnp.float32)
    m = jnp.max(logits, axis=-1, keepdims=True)
    ex = jnp.exp(logits - m)
    se = jnp.sum(ex, axis=-1, keepdims=True)
    p = ex / se
    lse = m + jnp.log(se)
    zs_ref[...] = jnp.broadcast_to(jnp.sum(lse * lse), (1, 1, 8))
    eidx = jax.lax.broadcasted_iota(jnp.int32, (BT, E), 1)
    rank = jnp.zeros((BT, E), jnp.int32)
    for j in range(E):
        pj = p[:, j:j + 1]
        rank = rank + jnp.where((pj > p) | ((pj == p) & (j < eidx)), 1, 0)
    sel = rank < K
    self32 = sel.astype(jnp.float32)
    gm = self32 * p
    g = gm / (jnp.sum(gm, axis=-1, keepdims=True) + 1e-6)
    slot = jnp.zeros((BT, E), jnp.int32)
    for j in range(E):
        sj = jnp.where(sel[:, j:j + 1], 1, 0)
        slot = slot + jnp.where(eidx > j, sj, 0)
    y_ref[...] = jnp.dot(x, wq_ref[...], preferred_element_type=jnp.float32)
    eidxf = eidx.astype(jnp.float32)
    i = pl.program_id(0)
    tgi = jax.lax.broadcasted_iota(jnp.int32, (BT, 1), 0) + i * BT
    tglob = tgi.astype(jnp.float32)
    cols = []
    for kk in range(K):
        sk = self32 * (slot == kk).astype(jnp.float32)
        etk = jnp.sum(sk * eidxf, axis=-1, keepdims=True)
        cols.append(tglob * E + etk)
    sq = jnp.concatenate(cols, axis=-1).astype(jnp.int32)
    sq_ref[...] = jnp.transpose(sq, (1, 0))
    sz_ref[...] = jnp.where(sel, slot * T + tgi, tgi)
    g_ref[...] = g
    fr_ref[0] = jnp.sum(self32, axis=0, keepdims=True)
    ps_ref[0] = jnp.sum(p, axis=0, keepdims=True)


def _attn_moe_kernel(q_ref, k_ref, v_ref, o_ref):
    s = jax.lax.dot_general(q_ref[0], k_ref[0], (((1,), (1,)), ((), ())),
                            preferred_element_type=jnp.float32)
    e = jnp.exp(s)
    oa = jnp.dot(e, v_ref[0], preferred_element_type=jnp.float32)
    ow = oa[:, :HD] / oa[:, HD:HD + 1]
    o_ref[0] = jnp.concatenate(
        [ow, jnp.zeros((BQ, VW - HD), jnp.float32)], axis=-1)


def _out_proj_kernel(z_ref, g_ref, sel_ref, wo_ref, out_ref):
    wide = jnp.dot(g_ref[...], sel_ref[...],
                   preferred_element_type=jnp.float32)
    out_ref[...] = jnp.dot(wide * z_ref[...], wo_ref[...],
                           preferred_element_type=jnp.float32)


_NW = 32
_CH = 128


def _make_sc_gather(rows):
    rpw = rows // _NW
    nch = rpw // _CH

    @functools.partial(
        pl.kernel,
        mesh=plsc.VectorSubcoreMesh(core_axis_name="c", subcore_axis_name="s"),
        out_type=jax.ShapeDtypeStruct((rows, VW), jnp.float32),
        scratch_types=[
            pltpu.VMEM((_CH,), jnp.int32),
            pltpu.VMEM((_CH, VW), jnp.float32),
            pltpu.SemaphoreType.DMA,
        ],
    )
    def _sc_gather(table_hbm, idx_hbm, out_hbm, idx_v, rows_v, sem):
        wid = lax.axis_index("s") * 2 + lax.axis_index("c")
        base = wid * rpw

        def body(j, carry):
            off = base + j * _CH
            pltpu.sync_copy(idx_hbm.at[pl.ds(off, _CH)], idx_v)
            pltpu.async_copy(table_hbm.at[idx_v], rows_v, sem).wait()
            pltpu.sync_copy(rows_v, out_hbm.at[pl.ds(off, _CH)])
            return carry

        lax.fori_loop(0, nch, body, 0)

    return _sc_gather


_QROWS = K * T
_ZROWS = T * E
_sc_gather_q = _make_sc_gather(_QROWS)
_sc_gather_z = _make_sc_gather(_ZROWS)


def kernel(x, w_gate, Wq, kv_w, kv_b, W_out, task_bh):
    xf = x.reshape(T, DIM)
    wg = w_gate[task_bh]
    wq_flat = jnp.pad(jnp.transpose(Wq, (1, 0, 2)),
                      ((0, 0), (0, 0), (0, VW - HD))).reshape(DIM, EHP)
    wo_flat = W_out.reshape(EH, DIM)
    kvb2 = kv_b.reshape(1, 2 * HD)
    eye_e = jnp.eye(E, dtype=jnp.float32)
    selmat = jnp.repeat(eye_e, VW, axis=1).reshape(E, EHP)
    wo_pad = jnp.pad(W_out, ((0, 0), (0, VW - HD), (0, 0))).reshape(EHP, DIM)

    y, k_, v_, srcq, srcz, g2, fr, ps, zs = pl.pallas_call(
        _route_proj_kernel,
        grid=(NBT,),
        in_specs=[
            pl.BlockSpec((BT, DIM), lambda i: (i, 0)),
            pl.BlockSpec((DIM, E), lambda i: (0, 0)),
            pl.BlockSpec((DIM, EHP), lambda i: (0, 0)),
            pl.BlockSpec((DIM, 2 * HD), lambda i: (0, 0)),
            pl.BlockSpec((1, 2 * HD), lambda i: (0, 0)),
        ],
        out_specs=[
            pl.BlockSpec((BT, EHP), lambda i: (i, 0)),
            pl.BlockSpec((BT, VW), lambda i: (i, 0)),
            pl.BlockSpec((BT, VW), lambda i: (i, 0)),
            pl.BlockSpec((K, BT), lambda i: (0, i)),
            pl.BlockSpec((BT, E), lambda i: (i, 0)),
            pl.BlockSpec((BT, E), lambda i: (i, 0)),
            pl.BlockSpec((1, 1, E), lambda i: (i, 0, 0)),
            pl.BlockSpec((1, 1, E), lambda i: (i, 0, 0)),
            pl.BlockSpec((1, 1, 8), lambda i: (i, 0, 0)),
        ],
        out_shape=[
            jax.ShapeDtypeStruct((T, EHP), jnp.float32),
            jax.ShapeDtypeStruct((T, VW), jnp.float32),
            jax.ShapeDtypeStruct((T, VW), jnp.float32),
            jax.ShapeDtypeStruct((K, T), jnp.int32),
            jax.ShapeDtypeStruct((T, E), jnp.int32),
            jax.ShapeDtypeStruct((T, E), jnp.float32),
            jax.ShapeDtypeStruct((NBT, 1, E), jnp.float32),
            jax.ShapeDtypeStruct((NBT, 1, E), jnp.float32),
            jax.ShapeDtypeStruct((NBT, 1, 8), jnp.float32),
        ],
    )(xf, wg, wq_flat, kv_w, kvb2)

    q = _sc_gather_q(y.reshape(T * E, VW), srcq.reshape(_QROWS))
    q = q.reshape(K, T, VW)

    o = pl.pallas_call(
        _attn_moe_kernel,
        grid=(B, NQ, K),
        in_specs=[
            pl.BlockSpec((1, BQ, VW), lambda b, i, h: (h, b * NQ + i, 0)),
            pl.BlockSpec((1, N, VW), lambda b, i, h: (0, b, 0)),
            pl.BlockSpec((1, N, VW), lambda b, i, h: (0, b, 0)),
        ],
        out_specs=pl.BlockSpec((1, BQ, VW), lambda b, i, h: (h, b * NQ + i, 0)),
        out_shape=jax.ShapeDtypeStruct((K, T, VW), jnp.float32),
    )(q, k_.reshape(1, T, VW), v_.reshape(1, T, VW))

    z = _sc_gather_z(o.reshape(K * T, VW), srcz.reshape(_ZROWS))

    out2d = pl.pallas_call(
        _out_proj_kernel,
        grid=(NBT,),
        in_specs=[
            pl.BlockSpec((BT, EHP), lambda i: (i, 0)),
            pl.BlockSpec((BT, E), lambda i: (i, 0)),
            pl.BlockSpec((E, EHP), lambda i: (0, 0)),
            pl.BlockSpec((EHP, DIM), lambda i: (0, 0)),
        ],
        out_specs=pl.BlockSpec((BT, DIM), lambda i: (i, 0)),
        out_shape=jax.ShapeDtypeStruct((T, DIM), jnp.float32),
    )(z.reshape(T, EHP), g2, selmat, wo_pad)

    out = out2d.reshape(B, N, DIM)

    zsum = jnp.sum(zs[:, 0, 0])
    zloss = 0.001 * zsum / T
    freqs = jnp.sum(fr[:, 0, :], axis=0)
    psum = jnp.sum(ps[:, 0, :], axis=0)
    freqs_n = freqs / (jnp.sum(freqs) + 1e-9)
    pm = psum / T
    pm_n = pm / (jnp.sum(pm) + 1e-9)
    switchloss = 0.1 * E * jnp.sum(pm_n * freqs_n)
    aux_loss = zloss + switchloss
    return out, aux_loss


# speedup vs baseline: 5.4523x; 1.1586x over previous
import functools

import jax
import jax.numpy as jnp
from jax import lax
from jax.experimental import pallas as pl
from jax.experimental.pallas import tpu as pltpu
from jax.experimental.pallas import tpu_sc as plsc

DIM = 768
E = 16
K = 8
HD = 96
B = 4
N = 2048
T = B * N
BT = 512
BQ = 512
NBT = T // BT
NQ = N // BQ
EH = E * HD
VW = 128
EHP = E * VW


def _route_proj_kernel(x_ref, wg_ref, wq_ref, kvw_ref, kvb_ref,
                       y_ref, k_ref, v_ref, ws_ref, sq_ref,
                       fr_ref, ps_ref, zs_ref):
    x = x_ref[...]
    kv = jnp.dot(x, kvw_ref[...], preferred_element_type=jnp.float32)
    kv = kv + kvb_ref[...]
    lane = jax.lax.broadcasted_iota(jnp.int32, (BT, VW - HD), 1)
    ones_pad = jnp.where(lane == 0, 1.0, 0.0)
    zeros_pad = jnp.zeros((BT, VW - HD), jnp.float32)
    k_ref[...] = jnp.concatenate([kv[:, :HD] * (HD ** -0.5), zeros_pad],
                                 axis=-1)
    v_ref[...] = jnp.concatenate([kv[:, HD:], ones_pad], axis=-1)
    logits = jnp.dot(x, wg_ref[...], preferred_element_type=jnp.float32)
    m = jnp.max(logits, axis=-1, keepdims=True)
    ex = jnp.exp(logits - m)
    se = jnp.sum(ex, axis=-1, keepdims=True)
    p = ex / se
    lse = m + jnp.log(se)
    zs_ref[...] = jnp.broadcast_to(jnp.sum(lse * lse), (1, 1, 8))
    eidx = jax.lax.broadcasted_iota(jnp.int32, (BT, E), 1)
    rank = jnp.zeros((BT, E), jnp.int32)
    for j in range(E):
        pj = p[:, j:j + 1]
        rank = rank + jnp.where((pj > p) | ((pj == p) & (j < eidx)), 1, 0)
    sel = rank < K
    self32 = sel.astype(jnp.float32)
    gm = self32 * p
    g = gm / (jnp.sum(gm, axis=-1, keepdims=True) + 1e-6)
    slot = jnp.zeros((BT, E), jnp.int32)
    for j in range(E):
        sj = jnp.where(sel[:, j:j + 1], 1, 0)
        slot = slot + jnp.where(eidx > j, sj, 0)
    y_ref[...] = jnp.dot(x, wq_ref[...], preferred_element_type=jnp.float32)
    eidxf = eidx.astype(jnp.float32)
    i = pl.program_id(0)
    tgi = jax.lax.broadcasted_iota(jnp.int32, (BT, 1), 0) + i * BT
    tglob = tgi.astype(jnp.float32)
    cols = []
    for kk in range(K):
        sk = self32 * (slot == kk).astype(jnp.float32)
        ws_ref[kk] = sk * g
        etk = jnp.sum(sk * eidxf, axis=-1, keepdims=True)
        cols.append(tglob * E + etk)
    sq = jnp.concatenate(cols, axis=-1).astype(jnp.int32)
    sq_ref[...] = jnp.transpose(sq, (1, 0))
    fr_ref[0] = jnp.sum(self32, axis=0, keepdims=True)
    ps_ref[0] = jnp.sum(p, axis=0, keepdims=True)


def _attn_moe_kernel(q_ref, k_ref, v_ref, o_ref):
    s = jax.lax.dot_general(q_ref[0], k_ref[0], (((1,), (1,)), ((), ())),
                            preferred_element_type=jnp.float32)
    e = jnp.exp(s)
    oa = jnp.dot(e, v_ref[0], preferred_element_type=jnp.float32)
    ow = oa[:, :HD] / oa[:, HD:HD + 1]
    o_ref[0] = jnp.concatenate(
        [ow, jnp.zeros((BQ, VW - HD), jnp.float32)], axis=-1)


def _reduce_kernel(o_ref, ws_ref, sel_ref, tile_ref, wo_ref, out_ref):
    selmat = sel_ref[...]
    tilem = tile_ref[...]
    z = jnp.zeros((BT, EH), jnp.float32)
    for kk in range(K):
        wide = jnp.dot(ws_ref[kk], selmat, preferred_element_type=jnp.float32)
        rep = jnp.dot(o_ref[kk][:, :HD], tilem,
                      preferred_element_type=jnp.float32)
        z = z + wide * rep
    out_ref[...] = jnp.dot(z, wo_ref[...], preferred_element_type=jnp.float32)


_NW = 32
_CH = 128


def _make_sc_gather(rows):
    rpw = rows // _NW
    nch = rpw // _CH

    @functools.partial(
        pl.kernel,
        mesh=plsc.VectorSubcoreMesh(core_axis_name="c", subcore_axis_name="s"),
        out_type=jax.ShapeDtypeStruct((rows, VW), jnp.float32),
        scratch_types=[
            pltpu.VMEM((_CH,), jnp.int32),
            pltpu.VMEM((_CH, VW), jnp.float32),
            pltpu.SemaphoreType.DMA,
        ],
    )
    def _sc_gather(table_hbm, idx_hbm, out_hbm, idx_v, rows_v, sem):
        wid = lax.axis_index("s") * 2 + lax.axis_index("c")
        base = wid * rpw

        def body(j, carry):
            off = base + j * _CH
            pltpu.sync_copy(idx_hbm.at[pl.ds(off, _CH)], idx_v)
            pltpu.async_copy(table_hbm.at[idx_v], rows_v, sem).wait()
            pltpu.sync_copy(rows_v, out_hbm.at[pl.ds(off, _CH)])
            return carry

        lax.fori_loop(0, nch, body, 0)

    return _sc_gather


_QROWS = K * T
_sc_gather_q = _make_sc_gather(_QROWS)


def kernel(x, w_gate, Wq, kv_w, kv_b, W_out, task_bh):
    xf = x.reshape(T, DIM)
    wg = w_gate[task_bh]
    wq_flat = jnp.pad(jnp.transpose(Wq, (1, 0, 2)),
                      ((0, 0), (0, 0), (0, VW - HD))).reshape(DIM, EHP)
    wo_flat = W_out.reshape(EH, DIM)
    kvb2 = kv_b.reshape(1, 2 * HD)
    eye_e = jnp.eye(E, dtype=jnp.float32)
    selmat = jnp.repeat(eye_e, HD, axis=1).reshape(E, EH)
    tilem = jnp.tile(jnp.eye(HD, dtype=jnp.float32), (1, E))

    y, k_, v_, ws, srcq, fr, ps, zs = pl.pallas_call(
        _route_proj_kernel,
        grid=(NBT,),
        in_specs=[
            pl.BlockSpec((BT, DIM), lambda i: (i, 0)),
            pl.BlockSpec((DIM, E), lambda i: (0, 0)),
            pl.BlockSpec((DIM, EHP), lambda i: (0, 0)),
            pl.BlockSpec((DIM, 2 * HD), lambda i: (0, 0)),
            pl.BlockSpec((1, 2 * HD), lambda i: (0, 0)),
        ],
        out_specs=[
            pl.BlockSpec((BT, EHP), lambda i: (i, 0)),
            pl.BlockSpec((BT, VW), lambda i: (i, 0)),
            pl.BlockSpec((BT, VW), lambda i: (i, 0)),
            pl.BlockSpec((K, BT, E), lambda i: (0, i, 0)),
            pl.BlockSpec((K, BT), lambda i: (0, i)),
            pl.BlockSpec((1, 1, E), lambda i: (i, 0, 0)),
            pl.BlockSpec((1, 1, E), lambda i: (i, 0, 0)),
            pl.BlockSpec((1, 1, 8), lambda i: (i, 0, 0)),
        ],
        out_shape=[
            jax.ShapeDtypeStruct((T, EHP), jnp.float32),
            jax.ShapeDtypeStruct((T, VW), jnp.float32),
            jax.ShapeDtypeStruct((T, VW), jnp.float32),
            jax.ShapeDtypeStruct((K, T, E), jnp.float32),
            jax.ShapeDtypeStruct((K, T), jnp.int32),
            jax.ShapeDtypeStruct((NBT, 1, E), jnp.float32),
            jax.ShapeDtypeStruct((NBT, 1, E), jnp.float32),
            jax.ShapeDtypeStruct((NBT, 1, 8), jnp.float32),
        ],
    )(xf, wg, wq_flat, kv_w, kvb2)

    q = _sc_gather_q(y.reshape(T * E, VW), srcq.reshape(_QROWS))
    q = q.reshape(K, T, VW)

    o = pl.pallas_call(
        _attn_moe_kernel,
        grid=(B, NQ, K),
        in_specs=[
            pl.BlockSpec((1, BQ, VW), lambda b, i, h: (h, b * NQ + i, 0)),
            pl.BlockSpec((1, N, VW), lambda b, i, h: (0, b, 0)),
            pl.BlockSpec((1, N, VW), lambda b, i, h: (0, b, 0)),
        ],
        out_specs=pl.BlockSpec((1, BQ, VW), lambda b, i, h: (h, b * NQ + i, 0)),
        out_shape=jax.ShapeDtypeStruct((K, T, VW), jnp.float32),
    )(q, k_.reshape(1, T, VW), v_.reshape(1, T, VW))

    out2d = pl.pallas_call(
        _reduce_kernel,
        grid=(NBT,),
        in_specs=[
            pl.BlockSpec((K, BT, VW), lambda i: (0, i, 0)),
            pl.BlockSpec((K, BT, E), lambda i: (0, i, 0)),
            pl.BlockSpec((E, EH), lambda i: (0, 0)),
            pl.BlockSpec((HD, EH), lambda i: (0, 0)),
            pl.BlockSpec((EH, DIM), lambda i: (0, 0)),
        ],
        out_specs=pl.BlockSpec((BT, DIM), lambda i: (i, 0)),
        out_shape=jax.ShapeDtypeStruct((T, DIM), jnp.float32),
    )(o, ws, selmat, tilem, wo_flat)

    out = out2d.reshape(B, N, DIM)

    zsum = jnp.sum(zs[:, 0, 0])
    zloss = 0.001 * zsum / T
    freqs = jnp.sum(fr[:, 0, :], axis=0)
    psum = jnp.sum(ps[:, 0, :], axis=0)
    freqs_n = freqs / (jnp.sum(freqs) + 1e-9)
    pm = psum / T
    pm_n = pm / (jnp.sum(pm) + 1e-9)
    switchloss = 0.1 * E * jnp.sum(pm_n * freqs_n)
    aux_loss = zloss + switchloss
    return out, aux_loss


# speedup vs baseline: 5.7104x; 1.0473x over previous
import functools

import jax
import jax.numpy as jnp
from jax import lax
from jax.experimental import pallas as pl
from jax.experimental.pallas import tpu as pltpu
from jax.experimental.pallas import tpu_sc as plsc

DIM = 768
E = 16
K = 8
HD = 96
B = 4
N = 2048
T = B * N
BT = 512
BQ = 1024
NBT = T // BT
NQ = N // BQ
EH = E * HD
VW = 128
EHP = E * VW


def _route_proj_kernel(x_ref, wg_ref, wq_ref, kvw_ref, kvb_ref,
                       y_ref, k_ref, v_ref, ws_ref, sq_ref,
                       fr_ref, ps_ref, zs_ref):
    x = x_ref[...]
    kv = jnp.dot(x, kvw_ref[...], preferred_element_type=jnp.float32)
    kv = kv + kvb_ref[...]
    lane = jax.lax.broadcasted_iota(jnp.int32, (BT, VW - HD), 1)
    ones_pad = jnp.where(lane == 0, 1.0, 0.0)
    zeros_pad = jnp.zeros((BT, VW - HD), jnp.float32)
    k_ref[...] = jnp.concatenate([kv[:, :HD] * (HD ** -0.5), zeros_pad],
                                 axis=-1)
    v_ref[...] = jnp.concatenate([kv[:, HD:], ones_pad], axis=-1)
    logits = jnp.dot(x, wg_ref[...], preferred_element_type=jnp.float32)
    m = jnp.max(logits, axis=-1, keepdims=True)
    ex = jnp.exp(logits - m)
    se = jnp.sum(ex, axis=-1, keepdims=True)
    p = ex / se
    lse = m + jnp.log(se)
    zs_ref[...] = jnp.broadcast_to(jnp.sum(lse * lse), (1, 1, 8))
    eidx = jax.lax.broadcasted_iota(jnp.int32, (BT, E), 1)
    rank = jnp.zeros((BT, E), jnp.int32)
    for j in range(E):
        pj = p[:, j:j + 1]
        rank = rank + jnp.where((pj > p) | ((pj == p) & (j < eidx)), 1, 0)
    sel = rank < K
    self32 = sel.astype(jnp.float32)
    gm = self32 * p
    g = gm / (jnp.sum(gm, axis=-1, keepdims=True) + 1e-6)
    slot = jnp.zeros((BT, E), jnp.int32)
    for j in range(E):
        sj = jnp.where(sel[:, j:j + 1], 1, 0)
        slot = slot + jnp.where(eidx > j, sj, 0)
    y_ref[...] = jnp.dot(x, wq_ref[...], preferred_element_type=jnp.float32)
    eidxf = eidx.astype(jnp.float32)
    i = pl.program_id(0)
    tgi = jax.lax.broadcasted_iota(jnp.int32, (BT, 1), 0) + i * BT
    tglob = tgi.astype(jnp.float32)
    cols = []
    for kk in range(K):
        sk = self32 * (slot == kk).astype(jnp.float32)
        ws_ref[kk] = sk * g
        etk = jnp.sum(sk * eidxf, axis=-1, keepdims=True)
        cols.append(tglob * E + etk)
    sq = jnp.concatenate(cols, axis=-1).astype(jnp.int32)
    sq_ref[...] = jnp.transpose(sq, (1, 0))
    fr_ref[0] = jnp.sum(self32, axis=0, keepdims=True)
    ps_ref[0] = jnp.sum(p, axis=0, keepdims=True)


def _attn_moe_kernel(q_ref, k_ref, v_ref, o_ref):
    s = jax.lax.dot_general(q_ref[0], k_ref[0], (((1,), (1,)), ((), ())),
                            preferred_element_type=jnp.float32)
    e = jnp.exp(s)
    oa = jnp.dot(e, v_ref[0], preferred_element_type=jnp.float32)
    ow = oa[:, :HD] / oa[:, HD:HD + 1]
    o_ref[0] = jnp.concatenate(
        [ow, jnp.zeros((BQ, VW - HD), jnp.float32)], axis=-1)


def _reduce_kernel(o_ref, ws_ref, sel_ref, tile_ref, wo_ref, out_ref):
    selmat = sel_ref[...]
    tilem = tile_ref[...]
    z = jnp.zeros((BT, EH), jnp.float32)
    for kk in range(K):
        wide = jnp.dot(ws_ref[kk], selmat, preferred_element_type=jnp.float32)
        rep = jnp.dot(o_ref[kk][:, :HD], tilem,
                      preferred_element_type=jnp.float32)
        z = z + wide * rep
    out_ref[...] = jnp.dot(z, wo_ref[...], preferred_element_type=jnp.float32)


_NW = 32
_CH = 128


def _make_sc_gather(rows):
    rpw = rows // _NW
    nch = rpw // _CH

    @functools.partial(
        pl.kernel,
        mesh=plsc.VectorSubcoreMesh(core_axis_name="c", subcore_axis_name="s"),
        out_type=jax.ShapeDtypeStruct((rows, VW), jnp.float32),
        scratch_types=[
            pltpu.VMEM((_CH,), jnp.int32),
            pltpu.VMEM((_CH, VW), jnp.float32),
            pltpu.SemaphoreType.DMA,
        ],
    )
    def _sc_gather(table_hbm, idx_hbm, out_hbm, idx_v, rows_v, sem):
        wid = lax.axis_index("s") * 2 + lax.axis_index("c")
        base = wid * rpw

        def body(j, carry):
            off = base + j * _CH
            pltpu.sync_copy(idx_hbm.at[pl.ds(off, _CH)], idx_v)
            pltpu.async_copy(table_hbm.at[idx_v], rows_v, sem).wait()
            pltpu.sync_copy(rows_v, out_hbm.at[pl.ds(off, _CH)])
            return carry

        lax.fori_loop(0, nch, body, 0)

    return _sc_gather


_QROWS = K * T
_sc_gather_q = _make_sc_gather(_QROWS)


def kernel(x, w_gate, Wq, kv_w, kv_b, W_out, task_bh):
    xf = x.reshape(T, DIM)
    wg = w_gate[task_bh]
    wq_flat = jnp.pad(jnp.transpose(Wq, (1, 0, 2)),
                      ((0, 0), (0, 0), (0, VW - HD))).reshape(DIM, EHP)
    wo_flat = W_out.reshape(EH, DIM)
    kvb2 = kv_b.reshape(1, 2 * HD)
    eye_e = jnp.eye(E, dtype=jnp.float32)
    selmat = jnp.repeat(eye_e, HD, axis=1).reshape(E, EH)
    tilem = jnp.tile(jnp.eye(HD, dtype=jnp.float32), (1, E))

    y, k_, v_, ws, srcq, fr, ps, zs = pl.pallas_call(
        _route_proj_kernel,
        grid=(NBT,),
        in_specs=[
            pl.BlockSpec((BT, DIM), lambda i: (i, 0)),
            pl.BlockSpec((DIM, E), lambda i: (0, 0)),
            pl.BlockSpec((DIM, EHP), lambda i: (0, 0)),
            pl.BlockSpec((DIM, 2 * HD), lambda i: (0, 0)),
            pl.BlockSpec((1, 2 * HD), lambda i: (0, 0)),
        ],
        out_specs=[
            pl.BlockSpec((BT, EHP), lambda i: (i, 0)),
            pl.BlockSpec((BT, VW), lambda i: (i, 0)),
            pl.BlockSpec((BT, VW), lambda i: (i, 0)),
            pl.BlockSpec((K, BT, E), lambda i: (0, i, 0)),
            pl.BlockSpec((K, BT), lambda i: (0, i)),
            pl.BlockSpec((1, 1, E), lambda i: (i, 0, 0)),
            pl.BlockSpec((1, 1, E), lambda i: (i, 0, 0)),
            pl.BlockSpec((1, 1, 8), lambda i: (i, 0, 0)),
        ],
        out_shape=[
            jax.ShapeDtypeStruct((T, EHP), jnp.float32),
            jax.ShapeDtypeStruct((T, VW), jnp.float32),
            jax.ShapeDtypeStruct((T, VW), jnp.float32),
            jax.ShapeDtypeStruct((K, T, E), jnp.float32),
            jax.ShapeDtypeStruct((K, T), jnp.int32),
            jax.ShapeDtypeStruct((NBT, 1, E), jnp.float32),
            jax.ShapeDtypeStruct((NBT, 1, E), jnp.float32),
            jax.ShapeDtypeStruct((NBT, 1, 8), jnp.float32),
        ],
    )(xf, wg, wq_flat, kv_w, kvb2)

    q = _sc_gather_q(y.reshape(T * E, VW), srcq.reshape(_QROWS))
    q = q.reshape(K, T, VW)

    o = pl.pallas_call(
        _attn_moe_kernel,
        grid=(B, NQ, K),
        in_specs=[
            pl.BlockSpec((1, BQ, VW), lambda b, i, h: (h, b * NQ + i, 0)),
            pl.BlockSpec((1, N, VW), lambda b, i, h: (0, b, 0)),
            pl.BlockSpec((1, N, VW), lambda b, i, h: (0, b, 0)),
        ],
        out_specs=pl.BlockSpec((1, BQ, VW), lambda b, i, h: (h, b * NQ + i, 0)),
        out_shape=jax.ShapeDtypeStruct((K, T, VW), jnp.float32),
    )(q, k_.reshape(1, T, VW), v_.reshape(1, T, VW))

    out2d = pl.pallas_call(
        _reduce_kernel,
        grid=(NBT,),
        in_specs=[
            pl.BlockSpec((K, BT, VW), lambda i: (0, i, 0)),
            pl.BlockSpec((K, BT, E), lambda i: (0, i, 0)),
            pl.BlockSpec((E, EH), lambda i: (0, 0)),
            pl.BlockSpec((HD, EH), lambda i: (0, 0)),
            pl.BlockSpec((EH, DIM), lambda i: (0, 0)),
        ],
        out_specs=pl.BlockSpec((BT, DIM), lambda i: (i, 0)),
        out_shape=jax.ShapeDtypeStruct((T, DIM), jnp.float32),
    )(o, ws, selmat, tilem, wo_flat)

    out = out2d.reshape(B, N, DIM)

    zsum = jnp.sum(zs[:, 0, 0])
    zloss = 0.001 * zsum / T
    freqs = jnp.sum(fr[:, 0, :], axis=0)
    psum = jnp.sum(ps[:, 0, :], axis=0)
    freqs_n = freqs / (jnp.sum(freqs) + 1e-9)
    pm = psum / T
    pm_n = pm / (jnp.sum(pm) + 1e-9)
    switchloss = 0.1 * E * jnp.sum(pm_n * freqs_n)
    aux_loss = zloss + switchloss
    return out, aux_loss
